# baseline (speedup 1.0000x reference)
import functools

import jax
import jax.numpy as jnp
from jax import lax
from jax.experimental import pallas as pl
from jax.experimental.pallas import tpu as pltpu
from jax.experimental.pallas import tpu_sc as plsc

BATCH = 16384
D = 64
M = 1000000
CB = 8192
HB = CB // 2
NBLK = (M + CB - 1) // CB
OUTR = NBLK * HB
NC = 2
NS = 16
NW = NC * NS
BPW = BATCH // NW
CHUNK = 128
NCHUNK = BPW // CHUNK


def _relayout_body(t_ref, o_ref):
    xt = t_ref[...].T
    o_ref[:, :D] = xt[:HB, :]
    o_ref[:, D:] = xt[HB:, :]


def _tc_relayout(t):
    return pl.pallas_call(
        _relayout_body,
        grid=(NBLK,),
        in_specs=[pl.BlockSpec((D, CB), lambda i: (0, i))],
        out_specs=pl.BlockSpec((HB, 2 * D), lambda i: (i, 0)),
        out_shape=jax.ShapeDtypeStruct((OUTR, 2 * D), jnp.float32),
    )(t)


NBLK_SC = NBLK - 1
TCOL = CB // 128


def _sc_relayout(t):
    mesh = plsc.VectorSubcoreMesh(core_axis_name="c", subcore_axis_name="s")

    @functools.partial(
        pl.kernel,
        mesh=mesh,
        out_type=jax.ShapeDtypeStruct((OUTR, 2 * D), jnp.float32),
        compiler_params=pltpu.CompilerParams(needs_layout_passes=False),
        scratch_types=[
            pltpu.VMEM((D, 129), jnp.float32),
            pltpu.VMEM((D, 129), jnp.float32),
            pltpu.VMEM((128, 2 * D), jnp.float32),
            pltpu.SemaphoreType.DMA,
        ],
    )
    def k(t_hbm, o_hbm, in0, in1, ob, sem):
        w = lax.axis_index("s") * NC + lax.axis_index("c")
        iota = lax.iota(jnp.int32, 16)
        fvecs = [c * 16 + iota for c in range(4)]

        @pl.loop(0, NBLK_SC)
        def _(i):
            c0 = (i * TCOL + w) * 128
            cp0 = pltpu.async_copy(
                t_hbm.at[:, pl.ds(c0, 128)], in0.at[:, pl.ds(0, 128)], sem)
            cp1 = pltpu.async_copy(
                t_hbm.at[:, pl.ds(c0 + HB, 128)], in1.at[:, pl.ds(0, 128)],
                sem)
            cp0.wait()
            cp1.wait()

            @pl.loop(0, 128, step=4)
            def _(uu):
                for du in range(4):
                    u = uu + du
                    uvec = jnp.full((16,), 0, jnp.int32) + u
                    for half, src in ((0, in0), (1, in1)):
                        for c in range(4):
                            val = plsc.load_gather(src, [fvecs[c], uvec])
                            ob[u, pl.ds(half * D + c * 16, 16)] = val

            pltpu.sync_copy(ob, o_hbm.at[pl.ds(i * HB + w * 128, 128), :])

    return k(t)


def _relayout_tail_body(t_ref, partial_ref, o_ref):
    del partial_ref
    _relayout_body(t_ref, o_ref)


def _tc_relayout_tail(t, partial):
    return pl.pallas_call(
        _relayout_tail_body,
        grid=(1,),
        in_specs=[pl.BlockSpec((D, CB), lambda i: (0, NBLK - 1)),
                  pl.BlockSpec(memory_space=pltpu.MemorySpace.HBM)],
        out_specs=pl.BlockSpec((HB, 2 * D), lambda i: (NBLK - 1, 0)),
        out_shape=jax.ShapeDtypeStruct((OUTR, 2 * D), jnp.float32),
        input_output_aliases={1: 0},
    )(t, partial)


def _sc_gather(n_idx, table, *idx_arrays):
    mesh = plsc.VectorSubcoreMesh(core_axis_name="c", subcore_axis_name="s")
    rows_t = jax.ShapeDtypeStruct((BATCH, 2 * D), jnp.float32)

    @functools.partial(
        pl.kernel,
        mesh=mesh,
        out_type=(rows_t,) * n_idx,
        scratch_types=[pltpu.VMEM((BPW,), jnp.int32)] * n_idx + [
            pltpu.VMEM((BPW, 2 * D), jnp.float32),
            pltpu.SemaphoreType.DMA,
        ],
    )
    def k(*refs):
        idx_hbm = refs[:n_idx]
        t_hbm = refs[n_idx]
        outs = refs[n_idx + 1: 2 * n_idx + 1]
        idx_v = refs[2 * n_idx + 1: 3 * n_idx + 1]
        rows = refs[3 * n_idx + 1]
        sem = refs[3 * n_idx + 2]
        wid = lax.axis_index("s") * NC + lax.axis_index("c")
        base = wid * BPW
        for ih, iv in zip(idx_hbm, idx_v):
            pltpu.sync_copy(ih.at[pl.ds(base, BPW)], iv)
        for iv, out in zip(idx_v, outs):
            copies = []
            for c in range(NCHUNK):
                sl = pl.ds(c * CHUNK, CHUNK)
                copies.append(
                    pltpu.async_copy(t_hbm.at[iv.at[sl]], rows.at[sl], sem))
            for cp in copies:
                cp.wait()
            pltpu.sync_copy(rows, out.at[pl.ds(base, BPW)])

    return k(*idx_arrays, table)


TC_ROWS = 2048


def _tc_body(u_ref, bi_ref, bj_ref, pu_ref, pbi_ref, pbj_ref, oi_ref, oj_ref):
    def half(rows_ref, par_ref):
        rows = rows_ref[...]
        return jnp.where(par_ref[...] == 0, rows[:, :D], rows[:, D:])
    u = half(u_ref, pu_ref)
    bi = half(bi_ref, pbi_ref)
    bj = half(bj_ref, pbj_ref)
    oi_ref[...] = jnp.sum(u * bi, axis=1, keepdims=True)
    oj_ref[...] = jnp.sum(u * bj, axis=1, keepdims=True)


def _tc_reduce(urows, birows, bjrows, pu, pbi, pbj):
    out_t = jax.ShapeDtypeStruct((BATCH, 1), jnp.float32)
    grid = (BATCH // TC_ROWS,)
    row_spec = pl.BlockSpec((TC_ROWS, 2 * D), lambda i: (i, 0))
    par_spec = pl.BlockSpec((TC_ROWS, 1), lambda i: (i, 0))
    out_spec = pl.BlockSpec((TC_ROWS, 1), lambda i: (i, 0))
    return pl.pallas_call(
        _tc_body,
        grid=grid,
        in_specs=[row_spec, row_spec, row_spec, par_spec, par_spec, par_spec],
        out_specs=(out_spec, out_spec),
        out_shape=(out_t, out_t),
    )(urows, birows, bjrows, pu, pbi, pbj)


_CBS = CB.bit_length() - 1
_HBS = HB.bit_length() - 1


def _phys_row(idx):
    return ((idx >> _CBS) << _HBS) + (idx & (HB - 1))


def _half_bit(idx):
    return (idx >> _HBS) & 1


@jax.jit
def kernel(user, business_i, business_j, embed_user, embed_business):
    ebt = embed_business.T
    eb2p = _sc_relayout(ebt)
    eu2 = _tc_relayout(embed_user.T)
    (urows,) = _sc_gather(1, eu2, _phys_row(user))
    eb2 = _tc_relayout_tail(ebt, eb2p)
    birows, bjrows = _sc_gather(
        2, eb2, _phys_row(business_i), _phys_row(business_j))
    pu = _half_bit(user).reshape(BATCH, 1)
    pbi = _half_bit(business_i).reshape(BATCH, 1)
    pbj = _half_bit(business_j).reshape(BATCH, 1)
    pi, pj = _tc_reduce(urows, birows, bjrows, pu, pbi, pbj)
    return pi.reshape(BATCH), pj.reshape(BATCH)


# speedup vs baseline: 3.5140x; 3.5140x over previous
import functools

import jax
import jax.numpy as jnp
from jax import lax
from jax.experimental import pallas as pl
from jax.experimental.pallas import tpu as pltpu
from jax.experimental.pallas import tpu_sc as plsc

BATCH = 16384
D = 64
M = 1000000
CB = 16384
HB = CB // 2
NBLK = (M + CB - 1) // CB
OUTR = NBLK * HB
NC = 2
NS = 16
NW = NC * NS
BPW = BATCH // NW
CHUNK = 128
NCHUNK = BPW // CHUNK


def _relayout_body(t_ref, o_ref):
    xt = t_ref[...].T
    o_ref[:, :D] = xt[:HB, :]
    o_ref[:, D:] = xt[HB:, :]


def _tc_relayout(t):
    return pl.pallas_call(
        _relayout_body,
        grid=(NBLK,),
        in_specs=[pl.BlockSpec((D, CB), lambda i: (0, i))],
        out_specs=pl.BlockSpec((HB, 2 * D), lambda i: (i, 0)),
        out_shape=jax.ShapeDtypeStruct((OUTR, 2 * D), jnp.float32),
    )(t)


NBLK_SC = NBLK - 1
TCOL = CB // 128


def _sc_relayout(t):
    mesh = plsc.VectorSubcoreMesh(core_axis_name="c", subcore_axis_name="s")

    @functools.partial(
        pl.kernel,
        mesh=mesh,
        out_type=jax.ShapeDtypeStruct((OUTR, 2 * D), jnp.float32),
        compiler_params=pltpu.CompilerParams(needs_layout_passes=False),
        scratch_types=[
            pltpu.VMEM((D, 129), jnp.float32),
            pltpu.VMEM((D, 129), jnp.float32),
            pltpu.VMEM((128, 2 * D), jnp.float32),
            pltpu.SemaphoreType.DMA,
        ],
    )
    def k(t_hbm, o_hbm, in0, in1, ob, sem):
        w = lax.axis_index("s") * NC + lax.axis_index("c")
        iota = lax.iota(jnp.int32, 16)
        fvecs = [c * 16 + iota for c in range(4)]

        @pl.loop(0, NBLK_SC)
        def _(i):
            c0 = (i * TCOL + w) * 128
            cp0 = pltpu.async_copy(
                t_hbm.at[:, pl.ds(c0, 128)], in0.at[:, pl.ds(0, 128)], sem)
            cp1 = pltpu.async_copy(
                t_hbm.at[:, pl.ds(c0 + HB, 128)], in1.at[:, pl.ds(0, 128)],
                sem)
            cp0.wait()
            cp1.wait()

            @pl.loop(0, 128, step=4)
            def _(uu):
                for du in range(4):
                    u = uu + du
                    uvec = jnp.full((16,), 0, jnp.int32) + u
                    for half, src in ((0, in0), (1, in1)):
                        for c in range(4):
                            val = plsc.load_gather(src, [fvecs[c], uvec])
                            ob[u, pl.ds(half * D + c * 16, 16)] = val

            pltpu.sync_copy(ob, o_hbm.at[pl.ds(i * HB + w * 128, 128), :])

    return k(t)


def _relayout_tail_body(t_ref, partial_ref, o_ref):
    del partial_ref
    _relayout_body(t_ref, o_ref)


def _tc_relayout_tail(t, partial):
    return pl.pallas_call(
        _relayout_tail_body,
        grid=(1,),
        in_specs=[pl.BlockSpec((D, CB), lambda i: (0, NBLK - 1)),
                  pl.BlockSpec(memory_space=pltpu.MemorySpace.HBM)],
        out_specs=pl.BlockSpec((HB, 2 * D), lambda i: (NBLK - 1, 0)),
        out_shape=jax.ShapeDtypeStruct((OUTR, 2 * D), jnp.float32),
        input_output_aliases={1: 0},
    )(t, partial)


def _sc_gather(n_idx, table, *idx_arrays):
    mesh = plsc.VectorSubcoreMesh(core_axis_name="c", subcore_axis_name="s")
    rows_t = jax.ShapeDtypeStruct((BATCH, 2 * D), jnp.float32)

    @functools.partial(
        pl.kernel,
        mesh=mesh,
        out_type=(rows_t,) * n_idx,
        scratch_types=[pltpu.VMEM((BPW,), jnp.int32)] * n_idx + [
            pltpu.VMEM((BPW, 2 * D), jnp.float32),
            pltpu.SemaphoreType.DMA,
        ],
    )
    def k(*refs):
        idx_hbm = refs[:n_idx]
        t_hbm = refs[n_idx]
        outs = refs[n_idx + 1: 2 * n_idx + 1]
        idx_v = refs[2 * n_idx + 1: 3 * n_idx + 1]
        rows = refs[3 * n_idx + 1]
        sem = refs[3 * n_idx + 2]
        wid = lax.axis_index("s") * NC + lax.axis_index("c")
        base = wid * BPW
        for ih, iv in zip(idx_hbm, idx_v):
            pltpu.sync_copy(ih.at[pl.ds(base, BPW)], iv)
        for iv, out in zip(idx_v, outs):
            copies = []
            for c in range(NCHUNK):
                sl = pl.ds(c * CHUNK, CHUNK)
                copies.append(
                    pltpu.async_copy(t_hbm.at[iv.at[sl]], rows.at[sl], sem))
            for cp in copies:
                cp.wait()
            pltpu.sync_copy(rows, out.at[pl.ds(base, BPW)])

    return k(*idx_arrays, table)


TC_ROWS = 2048


def _tc_body(u_ref, bi_ref, bj_ref, pu_ref, pbi_ref, pbj_ref, oi_ref, oj_ref):
    def half(rows_ref, par_ref):
        rows = rows_ref[...]
        return jnp.where(par_ref[...] == 0, rows[:, :D], rows[:, D:])
    u = half(u_ref, pu_ref)
    bi = half(bi_ref, pbi_ref)
    bj = half(bj_ref, pbj_ref)
    oi_ref[...] = jnp.sum(u * bi, axis=1, keepdims=True)
    oj_ref[...] = jnp.sum(u * bj, axis=1, keepdims=True)


def _tc_reduce(urows, birows, bjrows, pu, pbi, pbj):
    out_t = jax.ShapeDtypeStruct((BATCH, 1), jnp.float32)
    grid = (BATCH // TC_ROWS,)
    row_spec = pl.BlockSpec((TC_ROWS, 2 * D), lambda i: (i, 0))
    par_spec = pl.BlockSpec((TC_ROWS, 1), lambda i: (i, 0))
    out_spec = pl.BlockSpec((TC_ROWS, 1), lambda i: (i, 0))
    return pl.pallas_call(
        _tc_body,
        grid=grid,
        in_specs=[row_spec, row_spec, row_spec, par_spec, par_spec, par_spec],
        out_specs=(out_spec, out_spec),
        out_shape=(out_t, out_t),
    )(urows, birows, bjrows, pu, pbi, pbj)


_CBS = CB.bit_length() - 1
_HBS = HB.bit_length() - 1


def _phys_row(idx):
    return ((idx >> _CBS) << _HBS) + (idx & (HB - 1))


def _half_bit(idx):
    return (idx >> _HBS) & 1


@jax.jit
def kernel(user, business_i, business_j, embed_user, embed_business):
    eu2 = _tc_relayout(embed_user.T)
    (urows,) = _sc_gather(1, eu2, _phys_row(user))
    eb2 = _tc_relayout(embed_business.T)
    birows, bjrows = _sc_gather(
        2, eb2, _phys_row(business_i), _phys_row(business_j))
    pu = _half_bit(user).reshape(BATCH, 1)
    pbi = _half_bit(business_i).reshape(BATCH, 1)
    pbj = _half_bit(business_j).reshape(BATCH, 1)
    pi, pj = _tc_reduce(urows, birows, bjrows, pu, pbi, pbj)
    return pi.reshape(BATCH), pj.reshape(BATCH)


# speedup vs baseline: 3.7318x; 1.0620x over previous
import functools

import jax
import jax.numpy as jnp
from jax import lax
from jax.experimental import pallas as pl
from jax.experimental.pallas import tpu as pltpu
from jax.experimental.pallas import tpu_sc as plsc

BATCH = 16384
D = 64
M = 1000000
CB = 32768
HB = CB // 2
NBLK = (M + CB - 1) // CB
OUTR = NBLK * HB
NC = 2
NS = 16
NW = NC * NS
BPW = BATCH // NW
CHUNK = 128
NCHUNK = BPW // CHUNK


def _relayout_body(t_ref, o_ref):
    xt = t_ref[...].T
    o_ref[:, :D] = xt[:HB, :]
    o_ref[:, D:] = xt[HB:, :]


def _tc_relayout(t):
    return pl.pallas_call(
        _relayout_body,
        grid=(NBLK,),
        in_specs=[pl.BlockSpec((D, CB), lambda i: (0, i))],
        out_specs=pl.BlockSpec((HB, 2 * D), lambda i: (i, 0)),
        out_shape=jax.ShapeDtypeStruct((OUTR, 2 * D), jnp.float32),
    )(t)


NBLK_SC = NBLK - 1
TCOL = CB // 128


def _sc_relayout(t):
    mesh = plsc.VectorSubcoreMesh(core_axis_name="c", subcore_axis_name="s")

    @functools.partial(
        pl.kernel,
        mesh=mesh,
        out_type=jax.ShapeDtypeStruct((OUTR, 2 * D), jnp.float32),
        compiler_params=pltpu.CompilerParams(needs_layout_passes=False),
        scratch_types=[
            pltpu.VMEM((D, 129), jnp.float32),
            pltpu.VMEM((D, 129), jnp.float32),
            pltpu.VMEM((128, 2 * D), jnp.float32),
            pltpu.SemaphoreType.DMA,
        ],
    )
    def k(t_hbm, o_hbm, in0, in1, ob, sem):
        w = lax.axis_index("s") * NC + lax.axis_index("c")
        iota = lax.iota(jnp.int32, 16)
        fvecs = [c * 16 + iota for c in range(4)]

        @pl.loop(0, NBLK_SC)
        def _(i):
            c0 = (i * TCOL + w) * 128
            cp0 = pltpu.async_copy(
                t_hbm.at[:, pl.ds(c0, 128)], in0.at[:, pl.ds(0, 128)], sem)
            cp1 = pltpu.async_copy(
                t_hbm.at[:, pl.ds(c0 + HB, 128)], in1.at[:, pl.ds(0, 128)],
                sem)
            cp0.wait()
            cp1.wait()

            @pl.loop(0, 128, step=4)
            def _(uu):
                for du in range(4):
                    u = uu + du
                    uvec = jnp.full((16,), 0, jnp.int32) + u
                    for half, src in ((0, in0), (1, in1)):
                        for c in range(4):
                            val = plsc.load_gather(src, [fvecs[c], uvec])
                            ob[u, pl.ds(half * D + c * 16, 16)] = val

            pltpu.sync_copy(ob, o_hbm.at[pl.ds(i * HB + w * 128, 128), :])

    return k(t)


def _relayout_tail_body(t_ref, partial_ref, o_ref):
    del partial_ref
    _relayout_body(t_ref, o_ref)


def _tc_relayout_tail(t, partial):
    return pl.pallas_call(
        _relayout_tail_body,
        grid=(1,),
        in_specs=[pl.BlockSpec((D, CB), lambda i: (0, NBLK - 1)),
                  pl.BlockSpec(memory_space=pltpu.MemorySpace.HBM)],
        out_specs=pl.BlockSpec((HB, 2 * D), lambda i: (NBLK - 1, 0)),
        out_shape=jax.ShapeDtypeStruct((OUTR, 2 * D), jnp.float32),
        input_output_aliases={1: 0},
    )(t, partial)


def _sc_gather(n_idx, table, *idx_arrays):
    mesh = plsc.VectorSubcoreMesh(core_axis_name="c", subcore_axis_name="s")
    rows_t = jax.ShapeDtypeStruct((BATCH, 2 * D), jnp.float32)

    @functools.partial(
        pl.kernel,
        mesh=mesh,
        out_type=(rows_t,) * n_idx,
        scratch_types=[pltpu.VMEM((BPW,), jnp.int32)] * n_idx + [
            pltpu.VMEM((BPW, 2 * D), jnp.float32),
            pltpu.SemaphoreType.DMA,
        ],
    )
    def k(*refs):
        idx_hbm = refs[:n_idx]
        t_hbm = refs[n_idx]
        outs = refs[n_idx + 1: 2 * n_idx + 1]
        idx_v = refs[2 * n_idx + 1: 3 * n_idx + 1]
        rows = refs[3 * n_idx + 1]
        sem = refs[3 * n_idx + 2]
        wid = lax.axis_index("s") * NC + lax.axis_index("c")
        base = wid * BPW
        for ih, iv in zip(idx_hbm, idx_v):
            pltpu.sync_copy(ih.at[pl.ds(base, BPW)], iv)
        for iv, out in zip(idx_v, outs):
            copies = []
            for c in range(NCHUNK):
                sl = pl.ds(c * CHUNK, CHUNK)
                copies.append(
                    pltpu.async_copy(t_hbm.at[iv.at[sl]], rows.at[sl], sem))
            for cp in copies:
                cp.wait()
            pltpu.sync_copy(rows, out.at[pl.ds(base, BPW)])

    return k(*idx_arrays, table)


TC_ROWS = 2048


def _tc_body(u_ref, bi_ref, bj_ref, pu_ref, pbi_ref, pbj_ref, oi_ref, oj_ref):
    def half(rows_ref, par_ref):
        rows = rows_ref[...]
        return jnp.where(par_ref[...] == 0, rows[:, :D], rows[:, D:])
    u = half(u_ref, pu_ref)
    bi = half(bi_ref, pbi_ref)
    bj = half(bj_ref, pbj_ref)
    oi_ref[...] = jnp.sum(u * bi, axis=1, keepdims=True)
    oj_ref[...] = jnp.sum(u * bj, axis=1, keepdims=True)


def _tc_reduce(urows, birows, bjrows, pu, pbi, pbj):
    out_t = jax.ShapeDtypeStruct((BATCH, 1), jnp.float32)
    grid = (BATCH // TC_ROWS,)
    row_spec = pl.BlockSpec((TC_ROWS, 2 * D), lambda i: (i, 0))
    par_spec = pl.BlockSpec((TC_ROWS, 1), lambda i: (i, 0))
    out_spec = pl.BlockSpec((TC_ROWS, 1), lambda i: (i, 0))
    return pl.pallas_call(
        _tc_body,
        grid=grid,
        in_specs=[row_spec, row_spec, row_spec, par_spec, par_spec, par_spec],
        out_specs=(out_spec, out_spec),
        out_shape=(out_t, out_t),
    )(urows, birows, bjrows, pu, pbi, pbj)


_CBS = CB.bit_length() - 1
_HBS = HB.bit_length() - 1


def _phys_row(idx):
    return ((idx >> _CBS) << _HBS) + (idx & (HB - 1))


def _half_bit(idx):
    return (idx >> _HBS) & 1


@jax.jit
def kernel(user, business_i, business_j, embed_user, embed_business):
    eu2 = _tc_relayout(embed_user.T)
    (urows,) = _sc_gather(1, eu2, _phys_row(user))
    eb2 = _tc_relayout(embed_business.T)
    birows, bjrows = _sc_gather(
        2, eb2, _phys_row(business_i), _phys_row(business_j))
    pu = _half_bit(user).reshape(BATCH, 1)
    pbi = _half_bit(business_i).reshape(BATCH, 1)
    pbj = _half_bit(business_j).reshape(BATCH, 1)
    pi, pj = _tc_reduce(urows, birows, bjrows, pu, pbi, pbj)
    return pi.reshape(BATCH), pj.reshape(BATCH)
